# Initial kernel scaffold; baseline (speedup 1.0000x reference)
import jax
import jax.numpy as jnp
from jax.experimental import pallas as pl


def kernel(x, edge_index, edge_attr, W_edge, W_mlp, b_mlp):
    raise NotImplementedError("write your pallas kernel here")


# speedup vs baseline: 1.8652x; 1.8652x over previous
import jax
import jax.numpy as jnp
from jax import lax
from jax.experimental import pallas as pl
from jax.experimental.pallas import tpu as pltpu
from jax.experimental.pallas import tpu_sc as plsc

N = 10000
E = 160000
D = 300
DP = 512
CW = 128
NCH = DP // CW
DREM = D - 2 * CW
NB = 10
RB = N // NB
L = 16
NS = 16
NC = 2
SUB = 128
NSUB = E // SUB
S1 = NSUB // NS
NSUB2 = (E // NC) // SUB
S2 = NSUB2 // NS
RPT = 624
RREM = N - NS * RPT


def _prep_body(x_ref, wm_ref, b_ref, we_ref, h_ref, init_ref, w2_ref,
               hbuf, w2buf):
    j = pl.program_id(0)
    q = j % NCH

    @pl.when(q == 0)
    def _():
        wm = wm_ref[...]
        hb = lax.dot_general(x_ref[...], wm, (((1,), (1,)), ((), ())),
                             preferred_element_type=jnp.float32) + b_ref[...]
        w2 = lax.dot_general(we_ref[...], wm, (((1,), (1,)), ((), ())),
                             preferred_element_type=jnp.float32)
        hbuf[...] = jnp.pad(hb, ((0, 0), (0, DP - D)))
        w2buf[...] = jnp.pad(w2, ((0, 0), (0, DP - D)))

    def chunk_of(a):
        r = a[:, 3 * CW:]
        for c in range(NCH - 2, -1, -1):
            r = jnp.where(q == c, a[:, c * CW:(c + 1) * CW], r)
        return r

    hh = chunk_of(hbuf[...])
    w2h = chunk_of(w2buf[...])
    h_ref[...] = hh
    init_ref[...] = jnp.maximum(hh + 4.0 * w2h[1:2, :], 0.0)
    w2_ref[...] = jnp.pad(w2h, ((0, 6), (0, 0)))


def _prep(x, W_mlp, b_mlp, W_edge):
    return pl.pallas_call(
        _prep_body,
        grid=(NCH * NB,),
        in_specs=[
            pl.BlockSpec((RB, D), lambda j: (j // NCH, 0)),
            pl.BlockSpec((D, D), lambda j: (0, 0)),
            pl.BlockSpec((1, D), lambda j: (0, 0)),
            pl.BlockSpec((2, D), lambda j: (0, 0)),
        ],
        out_specs=[
            pl.BlockSpec((RB, CW), lambda j: ((j % NCH) * NB + j // NCH, 0)),
            pl.BlockSpec((RB, CW), lambda j: ((j % NCH) * NB + j // NCH, 0)),
            pl.BlockSpec((8, CW), lambda j: (j % NCH, 0)),
        ],
        out_shape=[
            jax.ShapeDtypeStruct((NCH * N, CW), jnp.float32),
            jax.ShapeDtypeStruct((NCH * N, CW), jnp.float32),
            jax.ShapeDtypeStruct((NCH * 8, CW), jnp.float32),
        ],
        scratch_shapes=[
            pltpu.VMEM((RB, DP), jnp.float32),
            pltpu.VMEM((2, DP), jnp.float32),
        ],
    )(x, W_mlp, b_mlp.reshape(1, D), W_edge)


def _sc_body(src_hbm, dst_hbm, ea0_hbm, ea1_hbm, h_hbm, init_hbm, w2_hbm,
             out1_hbm, out2_hbm,
             sidx_v, didx_v, ea0_v, ea1_v, rows_v, w2_v, acc, sem):
    cid = lax.axis_index("c")
    sid = lax.axis_index("s")

    def seed(init_row):
        pltpu.sync_copy(init_hbm.at[pl.ds(init_row + sid * RPT, RPT)],
                        acc.at[pl.ds(sid * RPT, RPT)])

        @pl.when(sid == NS - 1)
        def _():
            pltpu.sync_copy(init_hbm.at[pl.ds(init_row + NS * RPT, RREM)],
                            acc.at[pl.ds(NS * RPT, RREM)])

    def drain(out_hbm):
        out_row = cid * N
        pltpu.sync_copy(acc.at[pl.ds(sid * RPT, RPT)],
                        out_hbm.at[pl.ds(out_row + sid * RPT, RPT)])

        @pl.when(sid == NS - 1)
        def _():
            pltpu.sync_copy(acc.at[pl.ds(NS * RPT, RREM)],
                            out_hbm.at[pl.ds(out_row + NS * RPT, RREM)])

    def do_edges(base, gather_off):
        pltpu.sync_copy(src_hbm.at[pl.ds(base, SUB)], sidx_v)
        pltpu.sync_copy(dst_hbm.at[pl.ds(base, SUB)], didx_v)
        pltpu.sync_copy(ea0_hbm.at[pl.ds(base, SUB)], ea0_v.at[pl.ds(0, SUB)])
        pltpu.sync_copy(ea1_hbm.at[pl.ds(base, SUB)], ea1_v.at[pl.ds(0, SUB)])
        for j in range(SUB // L):
            sidx_v[pl.ds(j * L, L)] = sidx_v[pl.ds(j * L, L)] + gather_off
        pltpu.async_copy(h_hbm.at[sidx_v], rows_v, sem).wait()

        def edge_body(e, c):
            a0 = ea0_v[pl.ds(e, L)][0]
            a1 = ea1_v[pl.ds(e, L)][0]
            for j in range(CW // L):
                sl = pl.ds(j * L, L)
                r = rows_v[e, sl]
                rows_v[e, sl] = jnp.maximum(
                    r + a0 * w2_v[0, sl] + a1 * w2_v[1, sl], 0.0)
            return c

        lax.fori_loop(0, SUB, edge_body, 0)
        pltpu.sync_copy(rows_v, acc.at[didx_v], add=True)

    def phase(gather_off, w2_off, init_row, edge0, nsub, out_hbm):
        seed(init_row)
        pltpu.sync_copy(w2_hbm.at[pl.ds(w2_off, 2)], w2_v)
        plsc.subcore_barrier()

        def batch_body(b, c):
            do_edges(edge0 + b * SUB, gather_off)
            return c

        lax.fori_loop(0, nsub, batch_body, 0)
        plsc.subcore_barrier()
        drain(out_hbm)

    sub0 = S1 * sid + jnp.maximum(sid - (NS - 2), 0)
    n1 = S1 + (sid >= NS - 2).astype(jnp.int32)
    phase(cid * N, cid * 8, cid * N, sub0 * SUB, n1, out1_hbm)
    n2 = S2 + (sid == NS - 1).astype(jnp.int32)
    phase(2 * N, 16, (2 + cid) * N,
          cid * (E // NC) + S2 * sid * SUB, n2, out2_hbm)


def _sc_call(src, dst, ea0, ea1, h_cat, init_cat, w2_cat):
    mesh = plsc.VectorSubcoreMesh(core_axis_name="c", subcore_axis_name="s")
    return pl.kernel(
        _sc_body,
        out_type=[
            jax.ShapeDtypeStruct((2 * N, CW), jnp.float32),
            jax.ShapeDtypeStruct((2 * N, CW), jnp.float32),
        ],
        mesh=mesh,
        scratch_types=[
            pltpu.VMEM((SUB,), jnp.int32),
            pltpu.VMEM((SUB,), jnp.int32),
            pltpu.VMEM((SUB + L,), jnp.float32),
            pltpu.VMEM((SUB + L,), jnp.float32),
            pltpu.VMEM((SUB, CW), jnp.float32),
            pltpu.VMEM((2, CW), jnp.float32),
            pltpu.VMEM_SHARED((N, CW), jnp.float32),
            pltpu.SemaphoreType.DMA,
        ],
    )(src, dst, ea0, ea1, h_cat, init_cat, w2_cat)


@jax.jit
def kernel(x, edge_index, edge_attr, W_edge, W_mlp, b_mlp):
    h_cat, init_cat, w2_cat = _prep(x, W_mlp, b_mlp, W_edge)
    out1, out2 = _sc_call(edge_index[0], edge_index[1],
                          edge_attr[:, 0], edge_attr[:, 1],
                          h_cat, init_cat, w2_cat)
    return jnp.concatenate(
        [out1[:N], out1[N:], (out2[:N] + out2[N:])[:, :DREM]], axis=1)


# speedup vs baseline: 7.0379x; 3.7732x over previous
import functools

import jax
import jax.numpy as jnp
from jax import lax
from jax.experimental import layout as jlayout
from jax.experimental import pallas as pl
from jax.experimental.pallas import tpu as pltpu
from jax.experimental.pallas import tpu_sc as plsc

N = 10000
E = 160000
D = 300
DP = 512
CW = 128
NCH = DP // CW
DREM = D - 2 * CW
L = 16
NS = 16
NC = 2
SUB = 64
S1 = (E // SUB) // NS
R1T = E // SUB - NS * S1
S2 = (E // NC // SUB) // NS
R2T = E // NC // SUB - NS * S2
RPT = 624
RREM = N - NS * RPT


def _prep_body(x_ref, wm_ref, b_ref, we_ref, h_ref, init_ref, w2_ref):
    wmq = wm_ref[...]
    hq = lax.dot_general(x_ref[...], wmq, (((0,), (1,)), ((), ())),
                         preferred_element_type=jnp.float32) + b_ref[...]
    w2q = lax.dot_general(we_ref[...], wmq, (((1,), (1,)), ((), ())),
                          preferred_element_type=jnp.float32)
    h_ref[...] = hq
    init_ref[...] = jnp.maximum(hq + 4.0 * w2q[1:2, :], 0.0)
    w2_ref[...] = jnp.pad(w2q, ((0, 6), (0, 0)))


def _prep(x, W_mlp, b_mlp, W_edge):
    b_pad = jnp.pad(b_mlp, (0, DP - D)).reshape(1, DP)
    wm_pad = jnp.pad(W_mlp, ((0, DP - D), (0, 0)))
    return pl.pallas_call(
        _prep_body,
        grid=(NCH,),
        in_specs=[
            pl.BlockSpec((D, N), lambda q: (0, 0)),
            pl.BlockSpec((CW, D), lambda q: (q, 0)),
            pl.BlockSpec((1, CW), lambda q: (0, q)),
            pl.BlockSpec((2, D), lambda q: (0, 0)),
        ],
        out_specs=[
            pl.BlockSpec((N, CW), lambda q: (q, 0)),
            pl.BlockSpec((N, CW), lambda q: (q, 0)),
            pl.BlockSpec((8, CW), lambda q: (q, 0)),
        ],
        out_shape=[
            jax.ShapeDtypeStruct((NCH * N, CW), jnp.float32),
            jax.ShapeDtypeStruct((NCH * N, CW), jnp.float32),
            jax.ShapeDtypeStruct((NCH * 8, CW), jnp.float32),
        ],
    )(x.T, wm_pad, b_pad, W_edge)


def _sc_body(src_hbm, dst_hbm, ea0_hbm, ea1_hbm, h_hbm, init_hbm, w2_hbm,
             out1_hbm, out2_hbm,
             sidx_all, didx, ea0b, ea1b, rows, w2_v, acc, gs, ds, es, ss):
    cid = lax.axis_index("c")
    sid = lax.axis_index("s")

    def seed(init_row):
        pltpu.sync_copy(init_hbm.at[pl.ds(init_row + sid * RPT, RPT)],
                        acc.at[pl.ds(sid * RPT, RPT)])

        @pl.when(sid == NS - 1)
        def _():
            pltpu.sync_copy(init_hbm.at[pl.ds(init_row + NS * RPT, RREM)],
                            acc.at[pl.ds(NS * RPT, RREM)])

    def drain(out_hbm):
        out_row = cid * N
        pltpu.sync_copy(acc.at[pl.ds(sid * RPT, RPT)],
                        out_hbm.at[pl.ds(out_row + sid * RPT, RPT)])

        @pl.when(sid == NS - 1)
        def _():
            pltpu.sync_copy(acc.at[pl.ds(NS * RPT, RREM)],
                            out_hbm.at[pl.ds(out_row + NS * RPT, RREM)])

    def phase(nsub, ebase, goff, w2_row, init_row, out_hbm,
              rem_tiles, rem_base):
        assert nsub % 3 == 0
        seed(init_row)
        pltpu.sync_copy(w2_hbm.at[pl.ds(w2_row, 8)], w2_v)
        ne = nsub * SUB
        pltpu.sync_copy(src_hbm.at[pl.ds(ebase, ne)],
                        sidx_all.at[pl.ds(0, ne)])

        def add_off(k, c):
            sidx_all[pl.ds(k * L, L)] = sidx_all[pl.ds(k * L, L)] + goff
            return c

        lax.fori_loop(0, ne // L, add_off, 0)

        w0v = [w2_v[0, pl.ds(j * L, L)] for j in range(CW // L)]
        w1v = [w2_v[1, pl.ds(j * L, L)] for j in range(CW // L)]

        def compute(i):
            def edge_body(e, c):
                a0 = ea0b[i][pl.ds(e, L)][0]
                a1 = ea1b[i][pl.ds(e, L)][0]
                for j in range(CW // L):
                    sl = pl.ds(j * L, L)
                    rows[i][e, sl] = jnp.maximum(
                        rows[i][e, sl] + a0 * w0v[j] + a1 * w1v[j], 0.0)
                return c

            lax.fori_loop(0, SUB, edge_body, 0)

        def start_g(b, i):
            pltpu.async_copy(h_hbm.at[sidx_all.at[pl.ds(b * SUB, SUB)]],
                             rows[i], gs[i])
            pltpu.async_copy(dst_hbm.at[pl.ds(ebase + b * SUB, SUB)],
                             didx[i], ds[i])
            pltpu.async_copy(ea0_hbm.at[pl.ds(ebase + b * SUB, SUB)],
                             ea0b[i].at[pl.ds(0, SUB)], es[i])
            pltpu.async_copy(ea1_hbm.at[pl.ds(ebase + b * SUB, SUB)],
                             ea1b[i].at[pl.ds(0, SUB)], es[i])

        def wait_in(b, i):
            pltpu.make_async_copy(h_hbm.at[sidx_all.at[pl.ds(b * SUB, SUB)]],
                                  rows[i], gs[i]).wait()
            pltpu.make_async_copy(ea0_hbm.at[pl.ds(ebase + b * SUB, SUB)],
                                  ea0b[i].at[pl.ds(0, SUB)], es[i]).wait()
            pltpu.make_async_copy(ea1_hbm.at[pl.ds(ebase + b * SUB, SUB)],
                                  ea1b[i].at[pl.ds(0, SUB)], es[i]).wait()

        def wait_d(b, i):
            pltpu.make_async_copy(dst_hbm.at[pl.ds(ebase + b * SUB, SUB)],
                                  didx[i], ds[i]).wait()

        def scat_start(i):
            pltpu.async_copy(rows[i], acc.at[didx[i]], ss[i], add=True)

        def scat_wait(i):
            pltpu.make_async_copy(rows[i], acc.at[didx[i]], ss[i]).wait()

        plsc.subcore_barrier()

        start_g(0, 0)
        start_g(1, 1)

        def triple(p, c):
            b0 = 3 * p
            for i in range(3):
                b = b0 + i
                wait_in(b, i)
                compute(i)
                wait_d(b, i)
                scat_start(i)
                prev = (i + 2) % 3
                if i == 0:
                    @pl.when(p > 0)
                    def _():
                        scat_wait(prev)

                    start_g(b + 2, prev)
                else:
                    scat_wait(prev)

                    @pl.when(b + 2 < nsub)
                    def _():
                        start_g(b + 2, prev)
            return c

        lax.fori_loop(0, nsub // 3, triple, 0)
        scat_wait(2)

        @pl.when(sid < rem_tiles)
        def _():
            pltpu.sync_copy(src_hbm.at[pl.ds(rem_base, SUB)],
                            sidx_all.at[pl.ds(0, SUB)])
            pltpu.sync_copy(ea0_hbm.at[pl.ds(rem_base, SUB)],
                            ea0b[0].at[pl.ds(0, SUB)])
            pltpu.sync_copy(ea1_hbm.at[pl.ds(rem_base, SUB)],
                            ea1b[0].at[pl.ds(0, SUB)])
            lax.fori_loop(0, SUB // L, add_off, 0)
            pltpu.async_copy(h_hbm.at[sidx_all.at[pl.ds(0, SUB)]],
                             rows[0], gs[0]).wait()
            compute(0)
            pltpu.sync_copy(dst_hbm.at[pl.ds(rem_base, SUB)], didx[0])
            pltpu.sync_copy(rows[0], acc.at[didx[0]], add=True)

        plsc.subcore_barrier()
        drain(out_hbm)

    phase(S1, sid * (S1 * SUB), cid * N, cid * 8, cid * N, out1_hbm,
          R1T, (NS * S1 + sid) * SUB)
    half = cid * (E // NC)
    phase(S2, half + sid * (S2 * SUB), 2 * N, 16, (2 + cid) * N, out2_hbm,
          R2T, half + (NS * S2 + sid) * SUB)


def _sc_call(src, dst, ea0, ea1, h_cat, init_cat, w2_cat):
    mesh = plsc.VectorSubcoreMesh(core_axis_name="c", subcore_axis_name="s")
    return pl.kernel(
        _sc_body,
        out_type=[
            jax.ShapeDtypeStruct((2 * N, CW), jnp.float32),
            jax.ShapeDtypeStruct((2 * N, CW), jnp.float32),
        ],
        mesh=mesh,
        scratch_types=[
            pltpu.VMEM((S1 * SUB,), jnp.int32),
            [pltpu.VMEM((SUB,), jnp.int32) for _ in range(3)],
            [pltpu.VMEM((SUB + L,), jnp.float32) for _ in range(3)],
            [pltpu.VMEM((SUB + L,), jnp.float32) for _ in range(3)],
            [pltpu.VMEM((SUB, CW), jnp.float32) for _ in range(3)],
            pltpu.VMEM((8, CW), jnp.float32),
            pltpu.VMEM_SHARED((N, CW), jnp.float32),
            [pltpu.SemaphoreType.DMA for _ in range(3)],
            [pltpu.SemaphoreType.DMA for _ in range(3)],
            [pltpu.SemaphoreType.DMA for _ in range(3)],
            [pltpu.SemaphoreType.DMA for _ in range(3)],
        ],
    )(src, dst, ea0, ea1, h_cat, init_cat, w2_cat)


@jax.jit
def kernel(x, edge_index, edge_attr, W_edge, W_mlp, b_mlp):
    h_cat, init_cat, w2_cat = _prep(x, W_mlp, b_mlp, W_edge)
    out1, out2 = _sc_call(edge_index[0], edge_index[1],
                          edge_attr[:, 0], edge_attr[:, 1],
                          h_cat, init_cat, w2_cat)
    out_t = jnp.concatenate(
        [out1[:N].T, out1[N:].T, (out2[:N] + out2[N:])[:, :DREM].T], axis=0)
    return out_t.T
